# baseline (speedup 1.0000x reference)
import functools

import jax
import jax.numpy as jnp
from jax.experimental import pallas as pl

F32 = jnp.float32


def _conv_s2_body(x00, x01, x10, x11, w_ref, out_ref, *, S, relu):
    xs = ((x00, x01), (x10, x11))
    cin = x00.shape[-1]
    cout = w_ref.shape[-1]
    acc = jnp.zeros((8 * S * S, cout), F32)
    for kh in range(4):
        for kw in range(4):
            xr = xs[kh % 2][kw % 2]
            a, b = kh // 2, kw // 2
            patch = xr[:, a:a + S, b:b + S, :].reshape(8 * S * S, cin)
            acc = acc + jnp.dot(patch, w_ref[kh, kw],
                                preferred_element_type=F32)
    if relu:
        acc = jnp.maximum(acc, 0.0)
    out_ref[...] = acc


def _rb_body(xpad_ref, w3_ref, w1_ref, out_ref, *, S):
    C = xpad_ref.shape[-1]
    acc = jnp.zeros((8 * S * S, C), F32)
    for kh in range(3):
        for kw in range(3):
            patch = xpad_ref[:, kh:kh + S, kw:kw + S, :].reshape(8 * S * S, C)
            acc = acc + jnp.dot(patch, w3_ref[kh, kw],
                                preferred_element_type=F32)
    h = jnp.maximum(acc, 0.0)
    y = jnp.dot(h, w1_ref[...], preferred_element_type=F32)
    x_in = xpad_ref[:, 1:S + 1, 1:S + 1, :].reshape(8 * S * S, C)
    out_ref[...] = y + x_in


def _vq_body(z_ref, e_ref, et_ref, en_ref, zq_ref, loss_ref):
    z = z_ref[...]
    e = e_ref[...]
    s = en_ref[...] - 2.0 * jnp.dot(z, et_ref[...],
                                    preferred_element_type=F32)
    m = jnp.min(s, axis=1, keepdims=True)
    iota = jax.lax.broadcasted_iota(jnp.int32, s.shape, 1)
    idx = jnp.min(jnp.where(s == m, iota, s.shape[1]), axis=1)
    onehot = (iota == idx[:, None]).astype(F32)
    zq = jnp.dot(onehot, e, preferred_element_type=F32)
    zq_ref[...] = zq
    d = z - zq
    loss_ref[...] = jnp.sum(d * d, axis=(0, 1), keepdims=True) / 2048.0


def _deconv_body(xpad_ref, w_ref, b_ref, y00, y01, y10, y11, *, S, act):
    cin = xpad_ref.shape[-1]
    cout = w_ref.shape[-1]
    outs = ((y00, y01), (y10, y11))
    for r in (0, 1):
        for s in (0, 1):
            acc = jnp.zeros((8 * S * S, cout), F32) + b_ref[...]
            for a in (0, 1):
                for c in (0, 1):
                    patch = xpad_ref[:, r + a:r + a + S,
                                     s + c:s + c + S, :].reshape(8 * S * S, cin)
                    acc = acc + jnp.dot(patch, w_ref[2 * a + r, 2 * c + s],
                                        preferred_element_type=F32)
            outs[r][s][...] = act(acc)


def _call(body, out_shapes, *args):
    return pl.pallas_call(body, out_shape=out_shapes)(*args)


def _split_parity(xpad):
    return [xpad[:, p::2, q::2, :] for p in (0, 1) for q in (0, 1)]


def kernel(x, embed_w, enc_conv1_w, enc_conv2_w, enc_rb1_w1, enc_rb1_w2,
           enc_rb2_w1, enc_rb2_w2, dec_rb1_w1, dec_rb1_w2, dec_rb2_w1,
           dec_rb2_w2, dec_deconv1_w, dec_deconv1_b, dec_deconv2_w,
           dec_deconv2_b):
    B = x.shape[0]
    w1 = jnp.transpose(enc_conv1_w, (2, 3, 1, 0))
    w2 = jnp.transpose(enc_conv2_w, (2, 3, 1, 0))
    rb_w = [(jnp.transpose(w3, (2, 3, 1, 0)), jnp.transpose(wp[:, :, 0, 0]))
            for (w3, wp) in ((enc_rb1_w1, enc_rb1_w2), (enc_rb2_w1, enc_rb2_w2),
                             (dec_rb1_w1, dec_rb1_w2), (dec_rb2_w1, dec_rb2_w2))]
    wd1 = jnp.transpose(dec_deconv1_w[:, :, ::-1, ::-1], (2, 3, 0, 1))
    wd2 = jnp.transpose(dec_deconv2_w[:, :, ::-1, ::-1], (2, 3, 0, 1))
    b1 = dec_deconv1_b[None, :]
    b2 = dec_deconv2_b[None, :]

    x_nhwc = jnp.transpose(x, (0, 2, 3, 1))
    xp = jnp.pad(x_nhwc, ((0, 0), (1, 1), (1, 1), (0, 0)))
    z1 = _call(functools.partial(_conv_s2_body, S=32, relu=True),
               jax.ShapeDtypeStruct((B * 32 * 32, 64), F32),
               *_split_parity(xp), w1)
    z1 = z1.reshape(B, 32, 32, 64)
    z1p = jnp.pad(z1, ((0, 0), (1, 1), (1, 1), (0, 0)))
    z2 = _call(functools.partial(_conv_s2_body, S=16, relu=True),
               jax.ShapeDtypeStruct((B * 16 * 16, 64), F32),
               *_split_parity(z1p), w2)

    def res_block(flat, widx, S):
        t = flat.reshape(B, S, S, 64)
        tp = jnp.pad(t, ((0, 0), (1, 1), (1, 1), (0, 0)))
        return _call(functools.partial(_rb_body, S=S),
                     jax.ShapeDtypeStruct((B * S * S, 64), F32),
                     tp, rb_w[widx][0], rb_w[widx][1])

    z = res_block(z2, 0, 16)
    z = res_block(z, 1, 16)

    zq, loss = _call(_vq_body,
                     [jax.ShapeDtypeStruct((B * 16 * 16, 64), F32),
                      jax.ShapeDtypeStruct((1, 1), F32)],
                     z, embed_w, jnp.transpose(embed_w),
                     jnp.sum(embed_w * embed_w, axis=1)[None, :])

    d = res_block(zq, 2, 16)
    d = res_block(d, 3, 16)
    dp = jnp.pad(d.reshape(B, 16, 16, 64), ((0, 0), (1, 1), (1, 1), (0, 0)))
    ys = _call(functools.partial(_deconv_body, S=16,
                                 act=lambda v: jnp.maximum(v, 0.0)),
               [jax.ShapeDtypeStruct((B * 16 * 16, 64), F32)] * 4,
               dp, wd1, b1)
    u = jnp.stack(ys).reshape(2, 2, B, 16, 16, 64)
    u = jnp.transpose(u, (2, 3, 0, 4, 1, 5)).reshape(B, 32, 32, 64)
    up = jnp.pad(u, ((0, 0), (1, 1), (1, 1), (0, 0)))
    ys2 = _call(functools.partial(_deconv_body, S=32, act=jax.nn.sigmoid),
                [jax.ShapeDtypeStruct((B * 32 * 32, 3), F32)] * 4,
                up, wd2, b2)
    r = jnp.stack(ys2).reshape(2, 2, B, 32, 32, 3)
    r = jnp.transpose(r, (2, 3, 0, 4, 1, 5)).reshape(B, 64, 64, 3)
    recon = jnp.transpose(r, (0, 3, 1, 2))

    loss = loss.reshape(())
    return (recon, loss, loss)


# speedup vs baseline: 3.9903x; 3.9903x over previous
import functools

import jax
import jax.numpy as jnp
from jax.experimental import pallas as pl

F32 = jnp.float32


def _conv_s2_body(x00, x01, x10, x11, w_ref, out_ref, *, S, relu):
    xs = ((x00, x01), (x10, x11))
    cin = x00.shape[-1]
    cout = w_ref.shape[-1]
    acc = jnp.zeros((8 * S * S, cout), F32)
    for kh in range(4):
        for kw in range(4):
            xr = xs[kh % 2][kw % 2]
            a, b = kh // 2, kw // 2
            patch = xr[:, a:a + S, b:b + S, :].reshape(8 * S * S, cin)
            acc = acc + jnp.dot(patch, w_ref[kh, kw],
                                preferred_element_type=F32)
    if relu:
        acc = jnp.maximum(acc, 0.0)
    out_ref[...] = acc


def _rb_body(xpad_ref, w3_ref, w1_ref, out_ref, *, S):
    C = xpad_ref.shape[-1]
    acc = jnp.zeros((8 * S * S, C), F32)
    for kh in range(3):
        for kw in range(3):
            patch = xpad_ref[:, kh:kh + S, kw:kw + S, :].reshape(8 * S * S, C)
            acc = acc + jnp.dot(patch, w3_ref[kh, kw],
                                preferred_element_type=F32)
    h = jnp.maximum(acc, 0.0)
    y = jnp.dot(h, w1_ref[...], preferred_element_type=F32)
    x_in = xpad_ref[:, 1:S + 1, 1:S + 1, :].reshape(8 * S * S, C)
    out_ref[...] = y + x_in


def _vq_body(z_ref, e_ref, et_ref, en_ref, zq_ref, loss_ref):
    z = z_ref[...]
    e = e_ref[...]
    s = en_ref[...] - 2.0 * jnp.dot(z, et_ref[...],
                                    preferred_element_type=F32)
    m = jnp.min(s, axis=1, keepdims=True)
    iota = jax.lax.broadcasted_iota(jnp.int32, s.shape, 1)
    idx = jnp.min(jnp.where(s == m, iota, s.shape[1]), axis=1)
    onehot = (iota == idx[:, None]).astype(F32)
    zq = jnp.dot(onehot, e, preferred_element_type=F32)
    zq_ref[...] = zq
    d = z - zq
    loss_ref[...] = jnp.sum(d * d, axis=(0, 1), keepdims=True) / 2048.0


def _deconv_body(xpad_ref, w_ref, b_ref, y00, y01, y10, y11, *, S, act):
    cin = xpad_ref.shape[-1]
    cout = w_ref.shape[-1]
    outs = ((y00, y01), (y10, y11))
    for r in (0, 1):
        for s in (0, 1):
            acc = jnp.zeros((8 * S * S, cout), F32) + b_ref[...]
            for a in (0, 1):
                for c in (0, 1):
                    patch = xpad_ref[:, r + a:r + a + S,
                                     s + c:s + c + S, :].reshape(8 * S * S, cin)
                    acc = acc + jnp.dot(patch, w_ref[2 * a + r, 2 * c + s],
                                        preferred_element_type=F32)
            outs[r][s][...] = act(acc)


def _call(body, out_shapes, *args):
    return pl.pallas_call(body, out_shape=out_shapes)(*args)


def _split_parity(xpad):
    B, H, W, C = xpad.shape
    r = xpad.reshape(B, H // 2, 2, W // 2, 2, C)
    return [r[:, :, p, :, q, :] for p in (0, 1) for q in (0, 1)]


def kernel(x, embed_w, enc_conv1_w, enc_conv2_w, enc_rb1_w1, enc_rb1_w2,
           enc_rb2_w1, enc_rb2_w2, dec_rb1_w1, dec_rb1_w2, dec_rb2_w1,
           dec_rb2_w2, dec_deconv1_w, dec_deconv1_b, dec_deconv2_w,
           dec_deconv2_b):
    B = x.shape[0]
    w1 = jnp.transpose(enc_conv1_w, (2, 3, 1, 0))
    w2 = jnp.transpose(enc_conv2_w, (2, 3, 1, 0))
    rb_w = [(jnp.transpose(w3, (2, 3, 1, 0)), jnp.transpose(wp[:, :, 0, 0]))
            for (w3, wp) in ((enc_rb1_w1, enc_rb1_w2), (enc_rb2_w1, enc_rb2_w2),
                             (dec_rb1_w1, dec_rb1_w2), (dec_rb2_w1, dec_rb2_w2))]
    wd1 = jnp.transpose(dec_deconv1_w[:, :, ::-1, ::-1], (2, 3, 0, 1))
    wd2 = jnp.transpose(dec_deconv2_w[:, :, ::-1, ::-1], (2, 3, 0, 1))
    b1 = dec_deconv1_b[None, :]
    b2 = dec_deconv2_b[None, :]

    x_nhwc = jnp.transpose(x, (0, 2, 3, 1))
    xp = jnp.pad(x_nhwc, ((0, 0), (1, 1), (1, 1), (0, 0)))
    z1 = _call(functools.partial(_conv_s2_body, S=32, relu=True),
               jax.ShapeDtypeStruct((B * 32 * 32, 64), F32),
               *_split_parity(xp), w1)
    z1 = z1.reshape(B, 32, 32, 64)
    z1p = jnp.pad(z1, ((0, 0), (1, 1), (1, 1), (0, 0)))
    z2 = _call(functools.partial(_conv_s2_body, S=16, relu=True),
               jax.ShapeDtypeStruct((B * 16 * 16, 64), F32),
               *_split_parity(z1p), w2)

    def res_block(flat, widx, S):
        t = flat.reshape(B, S, S, 64)
        tp = jnp.pad(t, ((0, 0), (1, 1), (1, 1), (0, 0)))
        return _call(functools.partial(_rb_body, S=S),
                     jax.ShapeDtypeStruct((B * S * S, 64), F32),
                     tp, rb_w[widx][0], rb_w[widx][1])

    z = res_block(z2, 0, 16)
    z = res_block(z, 1, 16)

    zq, loss = _call(_vq_body,
                     [jax.ShapeDtypeStruct((B * 16 * 16, 64), F32),
                      jax.ShapeDtypeStruct((1, 1), F32)],
                     z, embed_w, jnp.transpose(embed_w),
                     jnp.sum(embed_w * embed_w, axis=1)[None, :])

    d = res_block(zq, 2, 16)
    d = res_block(d, 3, 16)
    dp = jnp.pad(d.reshape(B, 16, 16, 64), ((0, 0), (1, 1), (1, 1), (0, 0)))
    ys = _call(functools.partial(_deconv_body, S=16,
                                 act=lambda v: jnp.maximum(v, 0.0)),
               [jax.ShapeDtypeStruct((B * 16 * 16, 64), F32)] * 4,
               dp, wd1, b1)
    u = jnp.stack(ys).reshape(2, 2, B, 16, 16, 64)
    u = jnp.transpose(u, (2, 3, 0, 4, 1, 5)).reshape(B, 32, 32, 64)
    up = jnp.pad(u, ((0, 0), (1, 1), (1, 1), (0, 0)))
    ys2 = _call(functools.partial(_deconv_body, S=32, act=jax.nn.sigmoid),
                [jax.ShapeDtypeStruct((B * 32 * 32, 3), F32)] * 4,
                up, wd2, b2)
    r = jnp.stack(ys2).reshape(2, 2, B, 32, 32, 3)
    r = jnp.transpose(r, (2, 3, 0, 4, 1, 5)).reshape(B, 64, 64, 3)
    recon = jnp.transpose(r, (0, 3, 1, 2))

    loss = loss.reshape(())
    return (recon, loss, loss)
